# baseline (speedup 1.0000x reference)
import functools

import jax
import jax.numpy as jnp
from jax import lax
from jax.experimental import pallas as pl
from jax.experimental.pallas import tpu as pltpu
from jax.experimental.pallas import tpu_sc as plsc

_B = 16
_TOTAL = 16384
_D = 1024
_NC = 2
_NS = 16
_NW = _NC * _NS
_RPW = _TOTAL // _NW
_C = 32
_NCHUNK = _RPW // _C
_L = 16
_CUPAD = 24


def _sc_partials(flat, cu_pad):
    mesh = plsc.VectorSubcoreMesh(core_axis_name="c", subcore_axis_name="s")

    @functools.partial(
        pl.kernel,
        out_type=jax.ShapeDtypeStruct((_NW, _B, _D), jnp.float32),
        mesh=mesh,
        scratch_types=[
            pltpu.VMEM((_CUPAD,), jnp.int32),
            pltpu.VMEM((_C, _D), jnp.float32),
            pltpu.VMEM((_C, _D), jnp.float32),
            pltpu.VMEM((_B, _D), jnp.float32),
            pltpu.SemaphoreType.DMA,
            pltpu.SemaphoreType.DMA,
        ],
    )
    def body(flat_hbm, cu_hbm, out_hbm, cu_v, buf0, buf1, acc, sem0, sem1):
        wid = lax.axis_index("s") * _NC + lax.axis_index("c")
        row0 = wid * _RPW

        def chunk_src(cc):
            return flat_hbm.at[pl.ds(pl.multiple_of(row0 + cc * _C, _C), _C), :]

        pltpu.make_async_copy(chunk_src(0), buf0, sem0).start()
        pltpu.make_async_copy(chunk_src(1), buf1, sem1).start()
        pltpu.sync_copy(cu_hbm, cu_v)

        vlo = cu_v[pl.ds(0, _L)]
        vhi = cu_v[pl.ds(8, _L)]
        cu = [vlo[j] for j in range(_L)] + [vhi[j] for j in range(8, _L)]

        zero = jnp.zeros((_L,), jnp.float32)

        for j in range(_B):
            def zbody(g, _, j=j):
                acc[j, pl.ds(g * _L, _L)] = zero
                return 0

            lax.fori_loop(0, _D // _L, zbody, 0)

        def process(cc, buf):
            base = row0 + cc * _C
            ns = []
            for j in range(_B):
                lo = jnp.maximum(cu[j], base)
                hi = jnp.minimum(cu[j + 1], base + _C)
                ns.append(jnp.maximum(hi - lo, 0))
            is_full = ns[0] == _C
            s = jnp.int32(0)
            for j in range(_B):
                fj = ns[j] == _C
                is_full = jnp.logical_or(is_full, fj)
                s = s + jnp.where(fj, jnp.int32(j), jnp.int32(0))

            @pl.when(is_full)
            def _():
                def gbody(gg, _):
                    gbase = gg * (4 * _L)
                    a0 = acc[s, pl.ds(gbase + 0 * _L, _L)]
                    a1 = acc[s, pl.ds(gbase + 1 * _L, _L)]
                    a2 = acc[s, pl.ds(gbase + 2 * _L, _L)]
                    a3 = acc[s, pl.ds(gbase + 3 * _L, _L)]
                    for r in range(_C):
                        a0 = a0 + buf[r, pl.ds(gbase + 0 * _L, _L)]
                        a1 = a1 + buf[r, pl.ds(gbase + 1 * _L, _L)]
                        a2 = a2 + buf[r, pl.ds(gbase + 2 * _L, _L)]
                        a3 = a3 + buf[r, pl.ds(gbase + 3 * _L, _L)]
                    acc[s, pl.ds(gbase + 0 * _L, _L)] = a0
                    acc[s, pl.ds(gbase + 1 * _L, _L)] = a1
                    acc[s, pl.ds(gbase + 2 * _L, _L)] = a2
                    acc[s, pl.ds(gbase + 3 * _L, _L)] = a3
                    return 0

                lax.fori_loop(0, _D // (4 * _L), gbody, 0)

            @pl.when(jnp.logical_not(is_full))
            def _():
                _process_runs(ns, buf)

        def _process_runs(ns, buf):
            o = jnp.int32(0)
            for j in range(_B):
                n = ns[j]

                @pl.when(n > 0)
                def _(j=j, n=n, o=o):
                    def gbody(gg, _):
                        gbase = gg * (4 * _L)
                        a0 = acc[j, pl.ds(gbase + 0 * _L, _L)]
                        a1 = acc[j, pl.ds(gbase + 1 * _L, _L)]
                        a2 = acc[j, pl.ds(gbase + 2 * _L, _L)]
                        a3 = acc[j, pl.ds(gbase + 3 * _L, _L)]

                        def rbody(r, accs):
                            x0, x1, x2, x3 = accs
                            return (
                                x0 + buf[o + r, pl.ds(gbase + 0 * _L, _L)],
                                x1 + buf[o + r, pl.ds(gbase + 1 * _L, _L)],
                                x2 + buf[o + r, pl.ds(gbase + 2 * _L, _L)],
                                x3 + buf[o + r, pl.ds(gbase + 3 * _L, _L)],
                            )

                        a0, a1, a2, a3 = lax.fori_loop(
                            0, n, rbody, (a0, a1, a2, a3))
                        acc[j, pl.ds(gbase + 0 * _L, _L)] = a0
                        acc[j, pl.ds(gbase + 1 * _L, _L)] = a1
                        acc[j, pl.ds(gbase + 2 * _L, _L)] = a2
                        acc[j, pl.ds(gbase + 3 * _L, _L)] = a3
                        return 0

                    lax.fori_loop(0, _D // (4 * _L), gbody, 0)

                o = o + n

        def pair(i, _):
            cc = i * 2
            pltpu.make_async_copy(chunk_src(0), buf0, sem0).wait()
            process(cc, buf0)

            @pl.when(cc + 2 < _NCHUNK)
            def _():
                pltpu.make_async_copy(chunk_src(cc + 2), buf0, sem0).start()

            pltpu.make_async_copy(chunk_src(1), buf1, sem1).wait()
            process(cc + 1, buf1)

            @pl.when(cc + 3 < _NCHUNK)
            def _():
                pltpu.make_async_copy(chunk_src(cc + 3), buf1, sem1).start()

            return 0

        lax.fori_loop(0, _NCHUNK // 2, pair, 0)
        pltpu.sync_copy(acc, out_hbm.at[wid])

    return body(flat, cu_pad)


def _tc_reduce(partials):
    def body(p_ref, o_ref):
        o_ref[...] = jnp.sum(p_ref[...], axis=0)

    return pl.pallas_call(
        body,
        out_shape=jax.ShapeDtypeStruct((_B, _D), jnp.float32),
    )(partials)


def kernel(flat, cu_seqlens):
    cu_pad = jnp.concatenate(
        [cu_seqlens.astype(jnp.int32),
         jnp.zeros((_CUPAD - _B - 1,), jnp.int32)])
    partials = _sc_partials(flat, cu_pad)
    return _tc_reduce(partials)


# speedup vs baseline: 1.1930x; 1.1930x over previous
import functools

import jax
import jax.numpy as jnp
from jax import lax
from jax.experimental import pallas as pl
from jax.experimental.pallas import tpu as pltpu
from jax.experimental.pallas import tpu_sc as plsc

_B = 16
_TOTAL = 16384
_D = 1024
_NC = 2
_NS = 16
_NW = _NC * _NS
_S = 10240
_RPW = _S // _NW
_C = 32
_NCHUNK = _RPW // _C
_L = 16
_R = 512
_NT = (_TOTAL - _S) // _R


def _sc_partials(flat, cu_seqlens):
    mesh = plsc.VectorSubcoreMesh(core_axis_name="c", subcore_axis_name="s")

    @functools.partial(
        pl.kernel,
        out_type=jax.ShapeDtypeStruct((_NW, _B, _D), jnp.float32),
        mesh=mesh,
        scratch_types=[
            pltpu.VMEM((_B,), jnp.int32),
            pltpu.VMEM((_C, _D), jnp.float32),
            pltpu.VMEM((_C, _D), jnp.float32),
            pltpu.VMEM((_B, _D), jnp.float32),
            pltpu.SemaphoreType.DMA,
            pltpu.SemaphoreType.DMA,
        ],
    )
    def body(flat_hbm, cu_hbm, out_hbm, cu_v, buf0, buf1, acc, sem0, sem1):
        wid = lax.axis_index("s") * _NC + lax.axis_index("c")
        row0 = wid * _RPW

        def chunk_src(cc):
            return flat_hbm.at[pl.ds(pl.multiple_of(row0 + cc * _C, _C), _C), :]

        pltpu.make_async_copy(chunk_src(0), buf0, sem0).start()
        pltpu.make_async_copy(chunk_src(1), buf1, sem1).start()
        pltpu.sync_copy(cu_hbm.at[pl.ds(0, _B)], cu_v)

        vlo = cu_v[pl.ds(0, _L)]
        cu = [vlo[j] for j in range(_L)] + [jnp.int32(_TOTAL)]

        zero = jnp.zeros((_L,), jnp.float32)

        for j in range(_B):
            def zbody(g, _, j=j):
                acc[j, pl.ds(g * _L, _L)] = zero
                return 0

            lax.fori_loop(0, _D // _L, zbody, 0)

        def process(cc, buf):
            base = row0 + cc * _C
            ns = []
            for j in range(_B):
                lo = jnp.maximum(cu[j], base)
                hi = jnp.minimum(cu[j + 1], base + _C)
                ns.append(jnp.maximum(hi - lo, 0))
            is_full = ns[0] == _C
            s = jnp.int32(0)
            for j in range(_B):
                fj = ns[j] == _C
                is_full = jnp.logical_or(is_full, fj)
                s = s + jnp.where(fj, jnp.int32(j), jnp.int32(0))

            @pl.when(is_full)
            def _():
                def gbody(gg, _):
                    gbase = gg * (4 * _L)
                    a0 = acc[s, pl.ds(gbase + 0 * _L, _L)]
                    a1 = acc[s, pl.ds(gbase + 1 * _L, _L)]
                    a2 = acc[s, pl.ds(gbase + 2 * _L, _L)]
                    a3 = acc[s, pl.ds(gbase + 3 * _L, _L)]
                    for r in range(_C):
                        a0 = a0 + buf[r, pl.ds(gbase + 0 * _L, _L)]
                        a1 = a1 + buf[r, pl.ds(gbase + 1 * _L, _L)]
                        a2 = a2 + buf[r, pl.ds(gbase + 2 * _L, _L)]
                        a3 = a3 + buf[r, pl.ds(gbase + 3 * _L, _L)]
                    acc[s, pl.ds(gbase + 0 * _L, _L)] = a0
                    acc[s, pl.ds(gbase + 1 * _L, _L)] = a1
                    acc[s, pl.ds(gbase + 2 * _L, _L)] = a2
                    acc[s, pl.ds(gbase + 3 * _L, _L)] = a3
                    return 0

                lax.fori_loop(0, _D // (4 * _L), gbody, 0)

            @pl.when(jnp.logical_not(is_full))
            def _():
                _process_runs(ns, buf)

        def _process_runs(ns, buf):
            o = jnp.int32(0)
            for j in range(_B):
                n = ns[j]

                @pl.when(n > 0)
                def _(j=j, n=n, o=o):
                    def gbody(gg, _):
                        gbase = gg * (4 * _L)
                        a0 = acc[j, pl.ds(gbase + 0 * _L, _L)]
                        a1 = acc[j, pl.ds(gbase + 1 * _L, _L)]
                        a2 = acc[j, pl.ds(gbase + 2 * _L, _L)]
                        a3 = acc[j, pl.ds(gbase + 3 * _L, _L)]

                        def rbody(r, accs):
                            x0, x1, x2, x3 = accs
                            return (
                                x0 + buf[o + r, pl.ds(gbase + 0 * _L, _L)],
                                x1 + buf[o + r, pl.ds(gbase + 1 * _L, _L)],
                                x2 + buf[o + r, pl.ds(gbase + 2 * _L, _L)],
                                x3 + buf[o + r, pl.ds(gbase + 3 * _L, _L)],
                            )

                        a0, a1, a2, a3 = lax.fori_loop(
                            0, n, rbody, (a0, a1, a2, a3))
                        acc[j, pl.ds(gbase + 0 * _L, _L)] = a0
                        acc[j, pl.ds(gbase + 1 * _L, _L)] = a1
                        acc[j, pl.ds(gbase + 2 * _L, _L)] = a2
                        acc[j, pl.ds(gbase + 3 * _L, _L)] = a3
                        return 0

                    lax.fori_loop(0, _D // (4 * _L), gbody, 0)

                o = o + n

        def pair(i, _):
            cc = i * 2
            pltpu.make_async_copy(chunk_src(0), buf0, sem0).wait()
            process(cc, buf0)

            @pl.when(cc + 2 < _NCHUNK)
            def _():
                pltpu.make_async_copy(chunk_src(cc + 2), buf0, sem0).start()

            pltpu.make_async_copy(chunk_src(1), buf1, sem1).wait()
            process(cc + 1, buf1)

            @pl.when(cc + 3 < _NCHUNK)
            def _():
                pltpu.make_async_copy(chunk_src(cc + 3), buf1, sem1).start()

            return 0

        lax.fori_loop(0, _NCHUNK // 2, pair, 0)
        pltpu.sync_copy(acc, out_hbm.at[wid])

    return body(flat, cu_seqlens)


def _tc_tail_sums(flat, one_hot):
    def body(oh_ref, x_ref, o_ref):
        @pl.when(pl.program_id(0) == 0)
        def _():
            o_ref[...] = jnp.zeros_like(o_ref)

        o_ref[...] += lax.dot_general(
            oh_ref[...], x_ref[...],
            (((1,), (0,)), ((), ())),
            precision=lax.Precision.HIGHEST,
            preferred_element_type=jnp.float32)

    return pl.pallas_call(
        body,
        grid=(_NT,),
        in_specs=[
            pl.BlockSpec((_B, _R), lambda i: (0, i)),
            pl.BlockSpec((_R, _D), lambda i: (_S // _R + i, 0)),
        ],
        out_specs=pl.BlockSpec((_B, _D), lambda i: (0, 0)),
        out_shape=jax.ShapeDtypeStruct((_B, _D), jnp.float32),
    )(one_hot, flat)


def _fold(partials, tail):
    def body(p_ref, t_ref, o_ref):
        o_ref[...] = jnp.sum(p_ref[...], axis=0) + t_ref[...]

    return pl.pallas_call(
        body,
        out_shape=jax.ShapeDtypeStruct((_B, _D), jnp.float32),
    )(partials, tail)


def kernel(flat, cu_seqlens):
    cu = cu_seqlens.astype(jnp.int32)
    rows = jnp.arange(_S, _TOTAL, dtype=jnp.int32)[None, :]
    one_hot = ((rows >= cu[:-1, None]) & (rows < cu[1:, None])
               ).astype(jnp.float32)
    partials = _sc_partials(flat, cu)
    tail = _tc_tail_sums(flat, one_hot)
    return _fold(partials, tail)


# speedup vs baseline: 1.2373x; 1.0371x over previous
import functools

import jax
import jax.numpy as jnp
from jax import lax
from jax.experimental import pallas as pl
from jax.experimental.pallas import tpu as pltpu
from jax.experimental.pallas import tpu_sc as plsc

_B = 16
_TOTAL = 16384
_D = 1024
_NC = 2
_NS = 16
_NW = _NC * _NS
_S = 8192
_RPW = _S // _NW
_C = 32
_NCHUNK = _RPW // _C
_L = 16
_R = 512
_NT = (_TOTAL - _S) // _R


def _sc_partials(flat, cu_seqlens):
    mesh = plsc.VectorSubcoreMesh(core_axis_name="c", subcore_axis_name="s")

    @functools.partial(
        pl.kernel,
        out_type=jax.ShapeDtypeStruct((_NW, _B, _D), jnp.float32),
        mesh=mesh,
        scratch_types=[
            pltpu.VMEM((_B,), jnp.int32),
            pltpu.VMEM((_C, _D), jnp.float32),
            pltpu.VMEM((_C, _D), jnp.float32),
            pltpu.VMEM((_B, _D), jnp.float32),
            pltpu.SemaphoreType.DMA,
            pltpu.SemaphoreType.DMA,
        ],
    )
    def body(flat_hbm, cu_hbm, out_hbm, cu_v, buf0, buf1, acc, sem0, sem1):
        wid = lax.axis_index("s") * _NC + lax.axis_index("c")
        row0 = wid * _RPW

        def chunk_src(cc):
            return flat_hbm.at[pl.ds(pl.multiple_of(row0 + cc * _C, _C), _C), :]

        pltpu.make_async_copy(chunk_src(0), buf0, sem0).start()
        pltpu.make_async_copy(chunk_src(1), buf1, sem1).start()
        pltpu.sync_copy(cu_hbm.at[pl.ds(0, _B)], cu_v)

        vlo = cu_v[pl.ds(0, _L)]
        cu = [vlo[j] for j in range(_L)] + [jnp.int32(_TOTAL)]

        zero = jnp.zeros((_L,), jnp.float32)

        for j in range(_B):
            def zbody(g, _, j=j):
                acc[j, pl.ds(g * (4 * _L) + 0 * _L, _L)] = zero
                acc[j, pl.ds(g * (4 * _L) + 1 * _L, _L)] = zero
                acc[j, pl.ds(g * (4 * _L) + 2 * _L, _L)] = zero
                acc[j, pl.ds(g * (4 * _L) + 3 * _L, _L)] = zero
                return 0

            lax.fori_loop(0, _D // (4 * _L), zbody, 0)

        def process(cc, buf):
            base = row0 + cc * _C
            ns = []
            for j in range(_B):
                lo = jnp.maximum(cu[j], base)
                hi = jnp.minimum(cu[j + 1], base + _C)
                ns.append(jnp.maximum(hi - lo, 0))
            is_full = ns[0] == _C
            s = jnp.int32(0)
            for j in range(_B):
                fj = ns[j] == _C
                is_full = jnp.logical_or(is_full, fj)
                s = s + jnp.where(fj, jnp.int32(j), jnp.int32(0))

            @pl.when(is_full)
            def _():
                def gbody(gg, _):
                    gbase = gg * (4 * _L)
                    a0 = acc[s, pl.ds(gbase + 0 * _L, _L)]
                    a1 = acc[s, pl.ds(gbase + 1 * _L, _L)]
                    a2 = acc[s, pl.ds(gbase + 2 * _L, _L)]
                    a3 = acc[s, pl.ds(gbase + 3 * _L, _L)]
                    for r in range(_C):
                        a0 = a0 + buf[r, pl.ds(gbase + 0 * _L, _L)]
                        a1 = a1 + buf[r, pl.ds(gbase + 1 * _L, _L)]
                        a2 = a2 + buf[r, pl.ds(gbase + 2 * _L, _L)]
                        a3 = a3 + buf[r, pl.ds(gbase + 3 * _L, _L)]
                    acc[s, pl.ds(gbase + 0 * _L, _L)] = a0
                    acc[s, pl.ds(gbase + 1 * _L, _L)] = a1
                    acc[s, pl.ds(gbase + 2 * _L, _L)] = a2
                    acc[s, pl.ds(gbase + 3 * _L, _L)] = a3
                    return 0

                lax.fori_loop(0, _D // (4 * _L), gbody, 0)

            @pl.when(jnp.logical_not(is_full))
            def _():
                _process_runs(ns, buf)

        def _process_runs(ns, buf):
            o = jnp.int32(0)
            for j in range(_B):
                n = ns[j]

                @pl.when(n > 0)
                def _(j=j, n=n, o=o):
                    def gbody(gg, _):
                        gbase = gg * (4 * _L)
                        a0 = acc[j, pl.ds(gbase + 0 * _L, _L)]
                        a1 = acc[j, pl.ds(gbase + 1 * _L, _L)]
                        a2 = acc[j, pl.ds(gbase + 2 * _L, _L)]
                        a3 = acc[j, pl.ds(gbase + 3 * _L, _L)]

                        def rbody(r, accs):
                            x0, x1, x2, x3 = accs
                            return (
                                x0 + buf[o + r, pl.ds(gbase + 0 * _L, _L)],
                                x1 + buf[o + r, pl.ds(gbase + 1 * _L, _L)],
                                x2 + buf[o + r, pl.ds(gbase + 2 * _L, _L)],
                                x3 + buf[o + r, pl.ds(gbase + 3 * _L, _L)],
                            )

                        a0, a1, a2, a3 = lax.fori_loop(
                            0, n, rbody, (a0, a1, a2, a3))
                        acc[j, pl.ds(gbase + 0 * _L, _L)] = a0
                        acc[j, pl.ds(gbase + 1 * _L, _L)] = a1
                        acc[j, pl.ds(gbase + 2 * _L, _L)] = a2
                        acc[j, pl.ds(gbase + 3 * _L, _L)] = a3
                        return 0

                    lax.fori_loop(0, _D // (4 * _L), gbody, 0)

                o = o + n

        def pair(i, _):
            cc = i * 2
            pltpu.make_async_copy(chunk_src(0), buf0, sem0).wait()
            process(cc, buf0)

            @pl.when(cc + 2 < _NCHUNK)
            def _():
                pltpu.make_async_copy(chunk_src(cc + 2), buf0, sem0).start()

            pltpu.make_async_copy(chunk_src(1), buf1, sem1).wait()
            process(cc + 1, buf1)

            @pl.when(cc + 3 < _NCHUNK)
            def _():
                pltpu.make_async_copy(chunk_src(cc + 3), buf1, sem1).start()

            return 0

        lax.fori_loop(0, _NCHUNK // 2, pair, 0)
        pltpu.sync_copy(acc, out_hbm.at[wid])

    return body(flat, cu_seqlens)


def _tc_tail_sums(flat, one_hot):
    def body(oh_ref, x_ref, o_ref):
        @pl.when(pl.program_id(0) == 0)
        def _():
            o_ref[...] = jnp.zeros_like(o_ref)

        o_ref[...] += lax.dot_general(
            oh_ref[...], x_ref[...],
            (((1,), (0,)), ((), ())),
            precision=lax.Precision.HIGHEST,
            preferred_element_type=jnp.float32)

    return pl.pallas_call(
        body,
        grid=(_NT,),
        in_specs=[
            pl.BlockSpec((_B, _R), lambda i: (0, i)),
            pl.BlockSpec((_R, _D), lambda i: (_S // _R + i, 0)),
        ],
        out_specs=pl.BlockSpec((_B, _D), lambda i: (0, 0)),
        out_shape=jax.ShapeDtypeStruct((_B, _D), jnp.float32),
    )(one_hot, flat)


def _fold(partials, tail):
    def body(p_ref, t_ref, o_ref):
        o_ref[...] = jnp.sum(p_ref[...], axis=0) + t_ref[...]

    return pl.pallas_call(
        body,
        out_shape=jax.ShapeDtypeStruct((_B, _D), jnp.float32),
    )(partials, tail)


def kernel(flat, cu_seqlens):
    cu = cu_seqlens.astype(jnp.int32)
    rows = jnp.arange(_S, _TOTAL, dtype=jnp.int32)[None, :]
    one_hot = ((rows >= cu[:-1, None]) & (rows < cu[1:, None])
               ).astype(jnp.float32)
    partials = _sc_partials(flat, cu)
    tail = _tc_tail_sums(flat, one_hot)
    return _fold(partials, tail)


# speedup vs baseline: 1.2965x; 1.0478x over previous
import functools

import jax
import jax.numpy as jnp
from jax import lax
from jax.experimental import pallas as pl
from jax.experimental.pallas import tpu as pltpu
from jax.experimental.pallas import tpu_sc as plsc

_B = 16
_TOTAL = 16384
_D = 1024
_NC = 2
_NS = 16
_NW = _NC * _NS
_S = 8192
_RPW = _S // _NW
_C = 32
_NCHUNK = _RPW // _C
_L = 16
_R = 512
_NT = (_TOTAL - _S) // _R


def _sc_partials(flat, cu_seqlens):
    mesh = plsc.VectorSubcoreMesh(core_axis_name="c", subcore_axis_name="s")

    @functools.partial(
        pl.kernel,
        out_type=jax.ShapeDtypeStruct((_NW, _B, _D), jnp.float32),
        mesh=mesh,
        scratch_types=[
            pltpu.VMEM((_B,), jnp.int32),
            pltpu.SMEM((_B + 8,), jnp.int32),
            pltpu.VMEM((_C, _D), jnp.float32),
            pltpu.VMEM((_C, _D), jnp.float32),
            pltpu.VMEM((_B, _D), jnp.float32),
            pltpu.SemaphoreType.DMA,
            pltpu.SemaphoreType.DMA,
        ],
    )
    def body(flat_hbm, cu_hbm, out_hbm, cu_v, cu_s, buf0, buf1, acc,
             sem0, sem1):
        wid = lax.axis_index("s") * _NC + lax.axis_index("c")
        row0 = wid * _RPW

        def chunk_src(cc):
            return flat_hbm.at[pl.ds(pl.multiple_of(row0 + cc * _C, _C), _C), :]

        pltpu.make_async_copy(chunk_src(0), buf0, sem0).start()
        pltpu.make_async_copy(chunk_src(1), buf1, sem1).start()
        pltpu.sync_copy(cu_hbm.at[pl.ds(0, _B)], cu_v)

        vlo = cu_v[pl.ds(0, _L)]
        for j in range(_L):
            cu_s[j] = vlo[j]
        cu_s[_B] = jnp.int32(_TOTAL)

        zero = jnp.zeros((_L,), jnp.float32)

        for j in range(_B):
            def zbody(g, _, j=j):
                acc[j, pl.ds(g * (4 * _L) + 0 * _L, _L)] = zero
                acc[j, pl.ds(g * (4 * _L) + 1 * _L, _L)] = zero
                acc[j, pl.ds(g * (4 * _L) + 2 * _L, _L)] = zero
                acc[j, pl.ds(g * (4 * _L) + 3 * _L, _L)] = zero
                return 0

            lax.fori_loop(0, _D // (4 * _L), zbody, 0)

        def advance(j, pos):
            def step(_, jj):
                jn = jnp.minimum(jj + 1, _B)
                return jnp.where(cu_s[jn] <= pos, jn, jj)

            return lax.fori_loop(0, _B, step, j)

        def accum_run(buf, seg, o, n):
            def gbody(gg, _):
                gbase = gg * (4 * _L)
                a0 = acc[seg, pl.ds(gbase + 0 * _L, _L)]
                a1 = acc[seg, pl.ds(gbase + 1 * _L, _L)]
                a2 = acc[seg, pl.ds(gbase + 2 * _L, _L)]
                a3 = acc[seg, pl.ds(gbase + 3 * _L, _L)]

                def rbody(r, accs):
                    x0, x1, x2, x3 = accs
                    return (
                        x0 + buf[o + r, pl.ds(gbase + 0 * _L, _L)],
                        x1 + buf[o + r, pl.ds(gbase + 1 * _L, _L)],
                        x2 + buf[o + r, pl.ds(gbase + 2 * _L, _L)],
                        x3 + buf[o + r, pl.ds(gbase + 3 * _L, _L)],
                    )

                a0, a1, a2, a3 = lax.fori_loop(0, n, rbody, (a0, a1, a2, a3))
                acc[seg, pl.ds(gbase + 0 * _L, _L)] = a0
                acc[seg, pl.ds(gbase + 1 * _L, _L)] = a1
                acc[seg, pl.ds(gbase + 2 * _L, _L)] = a2
                acc[seg, pl.ds(gbase + 3 * _L, _L)] = a3
                return 0

            lax.fori_loop(0, _D // (4 * _L), gbody, 0)

        def process(cc, buf, jcur):
            base = row0 + cc * _C
            end = base + _C

            @pl.when(cu_s[jcur + 1] >= end)
            def _():
                def gbody(gg, _):
                    gbase = gg * (4 * _L)
                    a0 = acc[jcur, pl.ds(gbase + 0 * _L, _L)]
                    a1 = acc[jcur, pl.ds(gbase + 1 * _L, _L)]
                    a2 = acc[jcur, pl.ds(gbase + 2 * _L, _L)]
                    a3 = acc[jcur, pl.ds(gbase + 3 * _L, _L)]
                    for r in range(_C):
                        a0 = a0 + buf[r, pl.ds(gbase + 0 * _L, _L)]
                        a1 = a1 + buf[r, pl.ds(gbase + 1 * _L, _L)]
                        a2 = a2 + buf[r, pl.ds(gbase + 2 * _L, _L)]
                        a3 = a3 + buf[r, pl.ds(gbase + 3 * _L, _L)]
                    acc[jcur, pl.ds(gbase + 0 * _L, _L)] = a0
                    acc[jcur, pl.ds(gbase + 1 * _L, _L)] = a1
                    acc[jcur, pl.ds(gbase + 2 * _L, _L)] = a2
                    acc[jcur, pl.ds(gbase + 3 * _L, _L)] = a3
                    return 0

                lax.fori_loop(0, _D // (4 * _L), gbody, 0)

            @pl.when(cu_s[jcur + 1] < end)
            def _():
                def run(i, _):
                    j = jnp.minimum(jcur + i, _B - 1)
                    lo = jnp.maximum(cu_s[j], base)
                    hi = jnp.minimum(cu_s[j + 1], end)
                    n = jnp.maximum(hi - lo, 0)

                    @pl.when(jnp.logical_and(n > 0, jcur + i < _B))
                    def _():
                        accum_run(buf, j, lo - base, n)

                    return 0

                lax.fori_loop(0, _B, run, 0)

            return advance(jcur, end)

        def pair(i, jcur):
            cc = i * 2
            pltpu.make_async_copy(chunk_src(0), buf0, sem0).wait()
            jcur = process(cc, buf0, jcur)

            @pl.when(cc + 2 < _NCHUNK)
            def _():
                pltpu.make_async_copy(chunk_src(cc + 2), buf0, sem0).start()

            pltpu.make_async_copy(chunk_src(1), buf1, sem1).wait()
            jcur = process(cc + 1, buf1, jcur)

            @pl.when(cc + 3 < _NCHUNK)
            def _():
                pltpu.make_async_copy(chunk_src(cc + 3), buf1, sem1).start()

            return jcur

        lax.fori_loop(0, _NCHUNK // 2, pair, advance(jnp.int32(0), row0))
        pltpu.sync_copy(acc, out_hbm.at[wid])

    return body(flat, cu_seqlens)


def _tc_tail_sums(flat, one_hot):
    def body(oh_ref, x_ref, o_ref):
        @pl.when(pl.program_id(0) == 0)
        def _():
            o_ref[...] = jnp.zeros_like(o_ref)

        o_ref[...] += lax.dot_general(
            oh_ref[...], x_ref[...],
            (((1,), (0,)), ((), ())),
            precision=lax.Precision.HIGHEST,
            preferred_element_type=jnp.float32)

    return pl.pallas_call(
        body,
        grid=(_NT,),
        in_specs=[
            pl.BlockSpec((_B, _R), lambda i: (0, i)),
            pl.BlockSpec((_R, _D), lambda i: (_S // _R + i, 0)),
        ],
        out_specs=pl.BlockSpec((_B, _D), lambda i: (0, 0)),
        out_shape=jax.ShapeDtypeStruct((_B, _D), jnp.float32),
    )(one_hot, flat)


def _fold(partials, tail):
    def body(p_ref, t_ref, o_ref):
        o_ref[...] = jnp.sum(p_ref[...], axis=0) + t_ref[...]

    return pl.pallas_call(
        body,
        out_shape=jax.ShapeDtypeStruct((_B, _D), jnp.float32),
    )(partials, tail)


def kernel(flat, cu_seqlens):
    cu = cu_seqlens.astype(jnp.int32)
    rows = jnp.arange(_S, _TOTAL, dtype=jnp.int32)[None, :]
    one_hot = ((rows >= cu[:-1, None]) & (rows < cu[1:, None])
               ).astype(jnp.float32)
    partials = _sc_partials(flat, cu)
    tail = _tc_tail_sums(flat, one_hot)
    return _fold(partials, tail)


# speedup vs baseline: 1.3048x; 1.0064x over previous
import functools

import jax
import jax.numpy as jnp
from jax import lax
from jax.experimental import pallas as pl
from jax.experimental.pallas import tpu as pltpu
from jax.experimental.pallas import tpu_sc as plsc

_B = 16
_TOTAL = 16384
_D = 1024
_NC = 2
_NS = 16
_NW = _NC * _NS
_S = 8192
_RPW = _S // _NW
_C = 32
_NCHUNK = _RPW // _C
_L = 16
_R = 1024
_NT = (_TOTAL - _S) // _R


def _sc_partials(flat, cu_seqlens):
    mesh = plsc.VectorSubcoreMesh(core_axis_name="c", subcore_axis_name="s")

    @functools.partial(
        pl.kernel,
        out_type=jax.ShapeDtypeStruct((_NW, _B, _D), jnp.float32),
        mesh=mesh,
        scratch_types=[
            pltpu.VMEM((_B,), jnp.int32),
            pltpu.SMEM((_B + 8,), jnp.int32),
            pltpu.VMEM((_C, _D), jnp.float32),
            pltpu.VMEM((_C, _D), jnp.float32),
            pltpu.VMEM((_B, _D), jnp.float32),
            pltpu.SemaphoreType.DMA,
            pltpu.SemaphoreType.DMA,
        ],
    )
    def body(flat_hbm, cu_hbm, out_hbm, cu_v, cu_s, buf0, buf1, acc,
             sem0, sem1):
        wid = lax.axis_index("s") * _NC + lax.axis_index("c")
        row0 = wid * _RPW

        def chunk_src(cc):
            return flat_hbm.at[pl.ds(pl.multiple_of(row0 + cc * _C, _C), _C), :]

        pltpu.make_async_copy(chunk_src(0), buf0, sem0).start()
        pltpu.make_async_copy(chunk_src(1), buf1, sem1).start()
        pltpu.sync_copy(cu_hbm.at[pl.ds(0, _B)], cu_v)

        vlo = cu_v[pl.ds(0, _L)]
        for j in range(_L):
            cu_s[j] = vlo[j]
        cu_s[_B] = jnp.int32(_TOTAL)

        zero = jnp.zeros((_L,), jnp.float32)

        for j in range(_B):
            def zbody(g, _, j=j):
                acc[j, pl.ds(g * (4 * _L) + 0 * _L, _L)] = zero
                acc[j, pl.ds(g * (4 * _L) + 1 * _L, _L)] = zero
                acc[j, pl.ds(g * (4 * _L) + 2 * _L, _L)] = zero
                acc[j, pl.ds(g * (4 * _L) + 3 * _L, _L)] = zero
                return 0

            lax.fori_loop(0, _D // (4 * _L), zbody, 0)

        def advance(j, pos):
            def step(_, jj):
                jn = jnp.minimum(jj + 1, _B)
                return jnp.where(cu_s[jn] <= pos, jn, jj)

            return lax.fori_loop(0, _B, step, j)

        def accum_run(buf, seg, o, n):
            def gbody(gg, _):
                gbase = gg * (4 * _L)
                a0 = acc[seg, pl.ds(gbase + 0 * _L, _L)]
                a1 = acc[seg, pl.ds(gbase + 1 * _L, _L)]
                a2 = acc[seg, pl.ds(gbase + 2 * _L, _L)]
                a3 = acc[seg, pl.ds(gbase + 3 * _L, _L)]

                def rbody(r, accs):
                    x0, x1, x2, x3 = accs
                    return (
                        x0 + buf[o + r, pl.ds(gbase + 0 * _L, _L)],
                        x1 + buf[o + r, pl.ds(gbase + 1 * _L, _L)],
                        x2 + buf[o + r, pl.ds(gbase + 2 * _L, _L)],
                        x3 + buf[o + r, pl.ds(gbase + 3 * _L, _L)],
                    )

                a0, a1, a2, a3 = lax.fori_loop(0, n, rbody, (a0, a1, a2, a3))
                acc[seg, pl.ds(gbase + 0 * _L, _L)] = a0
                acc[seg, pl.ds(gbase + 1 * _L, _L)] = a1
                acc[seg, pl.ds(gbase + 2 * _L, _L)] = a2
                acc[seg, pl.ds(gbase + 3 * _L, _L)] = a3
                return 0

            lax.fori_loop(0, _D // (4 * _L), gbody, 0)

        def process(cc, buf, jcur):
            base = row0 + cc * _C
            end = base + _C

            @pl.when(cu_s[jcur + 1] >= end)
            def _():
                def gbody(gg, _):
                    gbase = gg * (4 * _L)
                    a0 = acc[jcur, pl.ds(gbase + 0 * _L, _L)]
                    a1 = acc[jcur, pl.ds(gbase + 1 * _L, _L)]
                    a2 = acc[jcur, pl.ds(gbase + 2 * _L, _L)]
                    a3 = acc[jcur, pl.ds(gbase + 3 * _L, _L)]
                    for r in range(_C):
                        a0 = a0 + buf[r, pl.ds(gbase + 0 * _L, _L)]
                        a1 = a1 + buf[r, pl.ds(gbase + 1 * _L, _L)]
                        a2 = a2 + buf[r, pl.ds(gbase + 2 * _L, _L)]
                        a3 = a3 + buf[r, pl.ds(gbase + 3 * _L, _L)]
                    acc[jcur, pl.ds(gbase + 0 * _L, _L)] = a0
                    acc[jcur, pl.ds(gbase + 1 * _L, _L)] = a1
                    acc[jcur, pl.ds(gbase + 2 * _L, _L)] = a2
                    acc[jcur, pl.ds(gbase + 3 * _L, _L)] = a3
                    return 0

                lax.fori_loop(0, _D // (4 * _L), gbody, 0)

            @pl.when(cu_s[jcur + 1] < end)
            def _():
                def run(i, _):
                    j = jnp.minimum(jcur + i, _B - 1)
                    lo = jnp.maximum(cu_s[j], base)
                    hi = jnp.minimum(cu_s[j + 1], end)
                    n = jnp.maximum(hi - lo, 0)

                    @pl.when(jnp.logical_and(n > 0, jcur + i < _B))
                    def _():
                        accum_run(buf, j, lo - base, n)

                    return 0

                lax.fori_loop(0, _B, run, 0)

            return advance(jcur, end)

        def pair(i, jcur):
            cc = i * 2
            pltpu.make_async_copy(chunk_src(0), buf0, sem0).wait()
            jcur = process(cc, buf0, jcur)

            @pl.when(cc + 2 < _NCHUNK)
            def _():
                pltpu.make_async_copy(chunk_src(cc + 2), buf0, sem0).start()

            pltpu.make_async_copy(chunk_src(1), buf1, sem1).wait()
            jcur = process(cc + 1, buf1, jcur)

            @pl.when(cc + 3 < _NCHUNK)
            def _():
                pltpu.make_async_copy(chunk_src(cc + 3), buf1, sem1).start()

            return jcur

        lax.fori_loop(0, _NCHUNK // 2, pair, advance(jnp.int32(0), row0))
        pltpu.sync_copy(acc, out_hbm.at[wid])

    return body(flat, cu_seqlens)


def _tc_tail_sums(flat, one_hot):
    def body(oh_ref, x_ref, o_ref):
        @pl.when(pl.program_id(0) == 0)
        def _():
            o_ref[...] = jnp.zeros_like(o_ref)

        o_ref[...] += lax.dot_general(
            oh_ref[...], x_ref[...],
            (((1,), (0,)), ((), ())),
            precision=lax.Precision.HIGHEST,
            preferred_element_type=jnp.float32)

    return pl.pallas_call(
        body,
        grid=(_NT,),
        in_specs=[
            pl.BlockSpec((_B, _R), lambda i: (0, i)),
            pl.BlockSpec((_R, _D), lambda i: (_S // _R + i, 0)),
        ],
        out_specs=pl.BlockSpec((_B, _D), lambda i: (0, 0)),
        out_shape=jax.ShapeDtypeStruct((_B, _D), jnp.float32),
    )(one_hot, flat)


def _fold(partials, tail):
    def body(p_ref, t_ref, o_ref):
        o_ref[...] = jnp.sum(p_ref[...], axis=0) + t_ref[...]

    return pl.pallas_call(
        body,
        out_shape=jax.ShapeDtypeStruct((_B, _D), jnp.float32),
    )(partials, tail)


def kernel(flat, cu_seqlens):
    cu = cu_seqlens.astype(jnp.int32)
    rows = jnp.arange(_S, _TOTAL, dtype=jnp.int32)[None, :]
    one_hot = ((rows >= cu[:-1, None]) & (rows < cu[1:, None])
               ).astype(jnp.float32)
    partials = _sc_partials(flat, cu)
    tail = _tc_tail_sums(flat, one_hot)
    return _fold(partials, tail)
